# Initial kernel scaffold; baseline (speedup 1.0000x reference)
import functools

import jax
import jax.numpy as jnp
from jax import lax
from jax.experimental import pallas as pl
from jax.experimental.pallas import tpu as pltpu
from jax.experimental.pallas import tpu_sc as plsc

_F = 26
_V = 100000
_D = 32
_B = 16384
_NC = 13
_W = _F * _D + _NC

_NWORKERS = 32
_BPW = _B // _NWORKERS
_C = 128
_NCHUNK = _BPW // _C


def _emb_body(x_hbm, tab_hbm, out_hbm, xb, idxb, gath, cont, sem):
    cid = lax.axis_index("c")
    sid = lax.axis_index("s")
    wid = sid * 2 + cid
    base = wid * _BPW
    iota = lax.iota(jnp.int32, 16)

    def chunk_body(ci, carry):
        cbase = base + ci * _C
        pltpu.async_copy(x_hbm.at[pl.ds(cbase, _C)], xb, sem).wait()

        def ib(j, c):
            jv = j * 16 + iota
            fv = jv >> 7
            bv = jv & (_C - 1)
            vals = plsc.load_gather(xb, [bv, fv])
            idxb[pl.ds(j * 16, 16)] = vals + fv * _V
            return c

        lax.fori_loop(0, (_F * _C) // 16, ib, 0)

        def fb(f, c):
            rows = gath.at[pl.ds(f * _C, _C)]
            pltpu.async_copy(
                tab_hbm.at[idxb.at[pl.ds(f * _C, _C)]], rows, sem
            ).wait()
            pltpu.sync_copy(
                rows, out_hbm.at[pl.ds(cbase, _C), pl.ds(f * _D, _D)]
            )
            return c

        lax.fori_loop(0, _F, fb, 0)

        def cb(j, c):
            jv = j * 16 + iota
            bv = jv // _NC
            kv = jv % _NC
            vals = plsc.load_gather(xb, [bv, kv + _F])
            cont[pl.ds(j * 16, 16)] = vals.astype(jnp.float32)
            return c

        lax.fori_loop(0, (_C * _NC) // 16, cb, 0)
        pltpu.sync_copy(
            cont, out_hbm.at[pl.ds(cbase, _C), pl.ds(_F * _D, _NC)]
        )
        return carry

    lax.fori_loop(0, _NCHUNK, chunk_body, 0)


@jax.jit
def _emb_call(x32, tab):
    mesh = plsc.VectorSubcoreMesh(core_axis_name="c", subcore_axis_name="s")
    run = functools.partial(
        pl.kernel,
        mesh=mesh,
        out_type=jax.ShapeDtypeStruct((_B, _W), jnp.float32),
        scratch_types=[
            pltpu.VMEM((_C, _F + _NC), jnp.int32),
            pltpu.VMEM((_F * _C,), jnp.int32),
            pltpu.VMEM((_F * _C, _D), jnp.float32),
            pltpu.VMEM((_C * _NC,), jnp.float32),
            pltpu.SemaphoreType.DMA,
        ],
    )(_emb_body)
    return run(x32, tab)


def kernel(X, tables):
    x32 = X.astype(jnp.int32)
    tab = tables.reshape(_F * _V, _D)
    return _emb_call(x32, tab)


# speedup vs baseline: 1.1399x; 1.1399x over previous
import functools

import jax
import jax.numpy as jnp
from jax import lax
from jax.experimental import pallas as pl
from jax.experimental.pallas import tpu as pltpu
from jax.experimental.pallas import tpu_sc as plsc

_F = 26
_V = 100000
_D = 32
_B = 16384
_NC = 13
_W = _F * _D + _NC

_NWORKERS = 32
_BPW = _B // _NWORKERS
_C = 128
_NCHUNK = _BPW // _C


def _emb_body(x_hbm, tab_hbm, out_hbm, xb, idxb, gath, cont, sem):
    cid = lax.axis_index("c")
    sid = lax.axis_index("s")
    wid = sid * 2 + cid
    base = wid * _BPW
    iota = lax.iota(jnp.int32, 16)

    def chunk_body(ci, carry):
        cbase = base + ci * _C
        pltpu.async_copy(x_hbm.at[pl.ds(cbase, _C)], xb, sem).wait()

        def ib(j, c):
            jv = j * 16 + iota
            fv = jv >> 7
            bv = jv & (_C - 1)
            vals = plsc.load_gather(xb, [bv, fv])
            idxb[pl.ds(j * 16, 16)] = vals + fv * _V
            return c

        lax.fori_loop(0, (_F * _C) // 16, ib, 0)

        def fb(f, c):
            rows = gath.at[pl.ds(f * _C, _C)]
            pltpu.async_copy(
                tab_hbm.at[idxb.at[pl.ds(f * _C, _C)]], rows, sem
            ).wait()
            pltpu.sync_copy(
                rows, out_hbm.at[pl.ds(cbase, _C), pl.ds(f * _D, _D)]
            )
            return c

        lax.fori_loop(0, _F, fb, 0)

        def cb(j, c):
            jv = j * 16 + iota
            bv = jv // _NC
            kv = jv % _NC
            vals = plsc.load_gather(xb, [bv, kv + _F])
            plsc.store_scatter(cont, [bv, kv], vals.astype(jnp.float32))
            return c

        lax.fori_loop(0, (_C * _NC) // 16, cb, 0)
        pltpu.sync_copy(
            cont, out_hbm.at[pl.ds(cbase, _C), pl.ds(_F * _D, _NC)]
        )
        return carry

    lax.fori_loop(0, _NCHUNK, chunk_body, 0)


@jax.jit
def _emb_call(x32, tab):
    mesh = plsc.VectorSubcoreMesh(core_axis_name="c", subcore_axis_name="s")
    run = functools.partial(
        pl.kernel,
        mesh=mesh,
        out_type=jax.ShapeDtypeStruct((_B, _W), jnp.float32),
        scratch_types=[
            pltpu.VMEM((_C, _F + _NC), jnp.int32),
            pltpu.VMEM((_F * _C,), jnp.int32),
            pltpu.VMEM((_F * _C, _D), jnp.float32),
            pltpu.VMEM((_C, _NC), jnp.float32),
            pltpu.SemaphoreType.DMA,
        ],
        compiler_params=pltpu.CompilerParams(
            use_tc_tiling_on_sc=False, needs_layout_passes=False
        ),
    )(_emb_body)
    return run(x32, tab)


def kernel(X, tables):
    x32 = X.astype(jnp.int32)
    tab = tables.reshape(_F * _V, _D)
    return _emb_call(x32, tab)


# speedup vs baseline: 1.2094x; 1.0610x over previous
import functools

import jax
import jax.numpy as jnp
from jax import lax
from jax.experimental import pallas as pl
from jax.experimental.pallas import tpu as pltpu
from jax.experimental.pallas import tpu_sc as plsc

_F = 26
_V = 100000
_D = 32
_B = 16384
_NC = 13
_W = _F * _D + _NC

_NWORKERS = 32
_BPW = _B // _NWORKERS
_C = 128
_NCHUNK = _BPW // _C


def _emb_body(x_hbm, tab_hbm, out_hbm, xb, idxb, gath, cont, sem, sem_w):
    cid = lax.axis_index("c")
    sid = lax.axis_index("s")
    wid = sid * 2 + cid
    base = wid * _BPW
    iota = lax.iota(jnp.int32, 16)

    def chunk_body(ci, carry):
        cbase = base + ci * _C
        pltpu.async_copy(x_hbm.at[pl.ds(cbase, _C)], xb, sem).wait()

        def ib(j, c):
            jv = j * 16 + iota
            fv = jv >> 7
            bv = jv & (_C - 1)
            vals = plsc.load_gather(xb, [bv, fv])
            idxb[pl.ds(j * 16, 16)] = vals + fv * _V
            return c

        lax.fori_loop(0, (_F * _C) // 16, ib, 0)

        def fire(f, c):
            pltpu.async_copy(
                tab_hbm.at[idxb.at[pl.ds(f * _C, _C)]],
                gath.at[pl.ds(f * _C, _C)],
                sem,
            )
            return c

        lax.fori_loop(0, _F, fire, 0)

        def cb(j, c):
            jv = j * 16 + iota
            bv = jv // _NC
            kv = jv % _NC
            vals = plsc.load_gather(xb, [bv, kv + _F])
            plsc.store_scatter(cont, [bv, kv], vals.astype(jnp.float32))
            return c

        lax.fori_loop(0, (_C * _NC) // 16, cb, 0)

        pltpu.make_async_copy(tab_hbm.at[pl.ds(0, _F * _C)], gath, sem).wait()

        def wfire(f, c):
            pltpu.async_copy(
                gath.at[pl.ds(f * _C, _C)],
                out_hbm.at[pl.ds(cbase, _C), pl.ds(f * _D, _D)],
                sem_w,
            )
            return c

        lax.fori_loop(0, _F, wfire, 0)
        pltpu.async_copy(
            cont, out_hbm.at[pl.ds(cbase, _C), pl.ds(_F * _D, _NC)], sem_w
        )

        def wdrain(f, c):
            pltpu.make_async_copy(
                gath.at[pl.ds(f * _C, _C)],
                out_hbm.at[pl.ds(cbase, _C), pl.ds(f * _D, _D)],
                sem_w,
            ).wait()
            return c

        lax.fori_loop(0, _F, wdrain, 0)
        pltpu.make_async_copy(
            cont, out_hbm.at[pl.ds(cbase, _C), pl.ds(_F * _D, _NC)], sem_w
        ).wait()
        return carry

    lax.fori_loop(0, _NCHUNK, chunk_body, 0)


@jax.jit
def _emb_call(x32, tab):
    mesh = plsc.VectorSubcoreMesh(core_axis_name="c", subcore_axis_name="s")
    run = functools.partial(
        pl.kernel,
        mesh=mesh,
        out_type=jax.ShapeDtypeStruct((_B, _W), jnp.float32),
        scratch_types=[
            pltpu.VMEM((_C, _F + _NC), jnp.int32),
            pltpu.VMEM((_F * _C,), jnp.int32),
            pltpu.VMEM((_F * _C, _D), jnp.float32),
            pltpu.VMEM((_C, _NC), jnp.float32),
            pltpu.SemaphoreType.DMA,
            pltpu.SemaphoreType.DMA,
        ],
        compiler_params=pltpu.CompilerParams(
            use_tc_tiling_on_sc=False, needs_layout_passes=False
        ),
    )(_emb_body)
    return run(x32, tab)


def kernel(X, tables):
    x32 = X.astype(jnp.int32)
    tab = tables.reshape(_F * _V, _D)
    return _emb_call(x32, tab)


# speedup vs baseline: 1.2106x; 1.0010x over previous
import functools

import jax
import jax.numpy as jnp
from jax import lax
from jax.experimental import pallas as pl
from jax.experimental.pallas import tpu as pltpu
from jax.experimental.pallas import tpu_sc as plsc

_F = 26
_V = 100000
_D = 32
_B = 16384
_NC = 13
_W = _F * _D + _NC

_NWORKERS = 32
_BPW = _B // _NWORKERS
_C = 128
_NCHUNK = _BPW // _C


def _emb_body(x_hbm, tab_hbm, out_hbm, xb, idxb, gath, cont, sem, sem_w):
    cid = lax.axis_index("c")
    sid = lax.axis_index("s")
    wid = sid * 2 + cid
    base = wid * _BPW
    iota = lax.iota(jnp.int32, 16)

    def chunk_body(ci, carry):
        cbase = base + ci * _C
        pltpu.async_copy(x_hbm.at[pl.ds(cbase, _C)], xb, sem).wait()

        def ib(j, c):
            jv = j * 16 + iota
            fv = jv >> 7
            bv = jv & (_C - 1)
            vals = plsc.load_gather(xb, [bv, fv])
            idxb[pl.ds(j * 16, 16)] = vals
            return c

        lax.fori_loop(0, (_F * _C) // 16, ib, 0)

        def fire(f, c):
            pltpu.async_copy(
                tab_hbm.at[f].at[idxb.at[pl.ds(f * _C, _C)]],
                gath.at[pl.ds(f * _C, _C)],
                sem,
            )
            return c

        lax.fori_loop(0, _F, fire, 0)

        def cb(j, c):
            jv = j * 16 + iota
            bv = jv // _NC
            kv = jv % _NC
            vals = plsc.load_gather(xb, [bv, kv + _F])
            plsc.store_scatter(cont, [bv, kv], vals.astype(jnp.float32))
            return c

        lax.fori_loop(0, (_C * _NC) // 16, cb, 0)

        pltpu.make_async_copy(
            tab_hbm.at[0].at[pl.ds(0, _F * _C)], gath, sem
        ).wait()

        def wfire(f, c):
            pltpu.async_copy(
                gath.at[pl.ds(f * _C, _C)],
                out_hbm.at[pl.ds(cbase, _C), pl.ds(f * _D, _D)],
                sem_w,
            )
            return c

        lax.fori_loop(0, _F, wfire, 0)
        pltpu.async_copy(
            cont, out_hbm.at[pl.ds(cbase, _C), pl.ds(_F * _D, _NC)], sem_w
        )

        def wdrain(f, c):
            pltpu.make_async_copy(
                gath.at[pl.ds(f * _C, _C)],
                out_hbm.at[pl.ds(cbase, _C), pl.ds(f * _D, _D)],
                sem_w,
            ).wait()
            return c

        lax.fori_loop(0, _F, wdrain, 0)
        pltpu.make_async_copy(
            cont, out_hbm.at[pl.ds(cbase, _C), pl.ds(_F * _D, _NC)], sem_w
        ).wait()
        return carry

    lax.fori_loop(0, _NCHUNK, chunk_body, 0)


@jax.jit
def _emb_call(x32, tab):
    mesh = plsc.VectorSubcoreMesh(core_axis_name="c", subcore_axis_name="s")
    run = functools.partial(
        pl.kernel,
        mesh=mesh,
        out_type=jax.ShapeDtypeStruct((_B, _W), jnp.float32),
        scratch_types=[
            pltpu.VMEM((_C, _F + _NC), jnp.int32),
            pltpu.VMEM((_F * _C,), jnp.int32),
            pltpu.VMEM((_F * _C, _D), jnp.float32),
            pltpu.VMEM((_C, _NC), jnp.float32),
            pltpu.SemaphoreType.DMA,
            pltpu.SemaphoreType.DMA,
        ],
        compiler_params=pltpu.CompilerParams(
            use_tc_tiling_on_sc=False, needs_layout_passes=False
        ),
    )(_emb_body)
    return run(x32, tab)


def kernel(X, tables):
    x32 = X.astype(jnp.int32)
    return _emb_call(x32, tables)


# speedup vs baseline: 1.9121x; 1.5794x over previous
import functools

import jax
import jax.numpy as jnp
from jax import lax
from jax.experimental import pallas as pl
from jax.experimental.pallas import tpu as pltpu
from jax.experimental.pallas import tpu_sc as plsc

_F = 26
_V = 100000
_D = 32
_B = 16384
_NC = 13
_R = _F * _D + _NC

_NWORKERS = 32
_CHUNK = 4096
_NCHUNK = _B // _CHUNK


def _emb_body(xt_hbm, tab_hbm, out_hbm, trow, xrow, ob0, ob1, sem, sw0, sw1):
    cid = lax.axis_index("c")
    sid = lax.axis_index("s")
    wid = sid * 2 + cid
    start = wid * 26 + jnp.minimum(wid, 13)
    cnt = 26 + jnp.where(wid < 13, 1, 0)
    obufs = (ob0, ob1)
    swsems = (sw0, sw1)

    def row_body(i, carry):
        r = start + i
        is_emb = r < _F * _D
        f = r >> 5
        xid = jnp.where(is_emb, f, r - _F * _D + _F)
        need_x = jnp.logical_or(
            jnp.logical_or(i == 0, lax.rem(r, _D) == 0),
            jnp.logical_not(is_emb),
        )

        @pl.when(need_x)
        def _():
            pltpu.async_copy(xt_hbm.at[xid], xrow, sem).wait()

        @pl.when(is_emb)
        def _():
            e = r & (_D - 1)
            pltpu.async_copy(tab_hbm.at[f, e], trow, sem).wait()

        for c2 in range(_NCHUNK):
            ob = obufs[c2 % 2]
            sw = swsems[c2 % 2]
            drain = pltpu.make_async_copy(
                ob, out_hbm.at[r, pl.ds(c2 * _CHUNK, _CHUNK)], sw
            )
            if c2 >= 2:
                drain.wait()
            else:
                @pl.when(i > 0)
                def _(d=drain):
                    d.wait()

            base = c2 * _CHUNK

            @pl.when(is_emb)
            def _(ob=ob, base=base):
                def gs(j, c):
                    v = xrow[pl.ds(base + j * 16, 16)]
                    ob[pl.ds(j * 16, 16)] = plsc.load_gather(trow, [v])
                    return c

                lax.fori_loop(0, _CHUNK // 16, gs, 0)

            @pl.when(jnp.logical_not(is_emb))
            def _(ob=ob, base=base):
                def cs(j, c):
                    v = xrow[pl.ds(base + j * 16, 16)]
                    ob[pl.ds(j * 16, 16)] = v.astype(jnp.float32)
                    return c

                lax.fori_loop(0, _CHUNK // 16, cs, 0)

            pltpu.async_copy(
                ob, out_hbm.at[r, pl.ds(c2 * _CHUNK, _CHUNK)], sw
            )
        return carry

    lax.fori_loop(0, cnt, row_body, 0)

    r_last = start + cnt - 1
    for c2 in (2, 3):
        pltpu.make_async_copy(
            obufs[c2 % 2],
            out_hbm.at[r_last, pl.ds(c2 * _CHUNK, _CHUNK)],
            swsems[c2 % 2],
        ).wait()


@jax.jit
def _emb_call(xt, tabt):
    mesh = plsc.VectorSubcoreMesh(core_axis_name="c", subcore_axis_name="s")
    run = functools.partial(
        pl.kernel,
        mesh=mesh,
        out_type=jax.ShapeDtypeStruct((_R, _B), jnp.float32),
        scratch_types=[
            pltpu.VMEM((_V,), jnp.float32),
            pltpu.VMEM((_B,), jnp.int32),
            pltpu.VMEM((_CHUNK,), jnp.float32),
            pltpu.VMEM((_CHUNK,), jnp.float32),
            pltpu.SemaphoreType.DMA,
            pltpu.SemaphoreType.DMA,
            pltpu.SemaphoreType.DMA,
        ],
        compiler_params=pltpu.CompilerParams(
            use_tc_tiling_on_sc=False, needs_layout_passes=False
        ),
    )(_emb_body)
    return run(xt, tabt)


def kernel(X, tables):
    xt = X.astype(jnp.int32).T
    tabt = tables.transpose(0, 2, 1)
    return _emb_call(xt, tabt).T
